# Initial kernel scaffold; baseline (speedup 1.0000x reference)
import jax
import jax.numpy as jnp
from jax.experimental import pallas as pl


def kernel(feats, bidx):
    raise NotImplementedError("write your pallas kernel here")


# speedup vs baseline: 14.2031x; 14.2031x over previous
import functools

import jax
import jax.numpy as jnp
from jax import lax
from jax.experimental import pallas as pl
from jax.experimental.pallas import tpu as pltpu
from jax.experimental.pallas import tpu_sc as plsc

_N = 320000
_C = 128
_B = 8
_NC = 2
_NS = 16
_NW = _NC * _NS
_CHUNK = _N // _NW
_R = 200
_NBLK = _CHUNK // _R
_NVEC = _CHUNK // 16
_CG = _C // 16


def _sc_partial(feats, bidx):
    mesh = plsc.VectorSubcoreMesh(core_axis_name="c", subcore_axis_name="s")

    @functools.partial(
        pl.kernel,
        mesh=mesh,
        out_type=[
            jax.ShapeDtypeStruct((_NW, _B, _C), jnp.float32),
            jax.ShapeDtypeStruct((_NW, _B, _C), jnp.float32),
        ],
        scratch_types=[
            pltpu.VMEM((_R, _C), jnp.float32),
            pltpu.VMEM((_R, _C), jnp.float32),
            pltpu.VMEM((_CHUNK,), jnp.int32),
            pltpu.VMEM((_B, _C), jnp.float32),
            pltpu.VMEM((_B, _C), jnp.float32),
            pltpu.SemaphoreType.DMA,
            pltpu.SemaphoreType.DMA,
            pltpu.SemaphoreType.DMA,
        ],
    )
    def k(feats_hbm, bidx_hbm, psum_hbm, pcnt_hbm,
          buf0, buf1, bidx_v, acc, cntb, sem0, sem1, semi):
        wid = lax.axis_index("c") * _NS + lax.axis_index("s")
        base = wid * _CHUNK

        cpi = pltpu.async_copy(bidx_hbm.at[pl.ds(base, _CHUNK)], bidx_v, semi)
        pltpu.async_copy(feats_hbm.at[pl.ds(base, _R), :], buf0, sem0)
        pltpu.async_copy(feats_hbm.at[pl.ds(base + _R, _R), :], buf1, sem1)
        cpi.wait()

        def lower_bound(target):
            def body(_, lohi):
                lo, hi = lohi
                mid = jnp.minimum((lo + hi) // 2, _NVEC - 1)
                key = bidx_v[pl.ds(mid * 16, 16)][0]
                active = lo < hi
                right = jnp.logical_and(active, key < target)
                left = jnp.logical_and(active, key >= target)
                return (jnp.where(right, mid + 1, lo),
                        jnp.where(left, mid, hi))

            g, _ = lax.fori_loop(
                0, 10, body, (jnp.int32(0), jnp.int32(_NVEC)))
            gx = jnp.maximum(g, 1) - 1
            v = bidx_v[pl.ds(gx * 16, 16)]
            cnt = jnp.int32(0)
            for lane in range(16):
                cnt = cnt + jnp.where(v[lane] < target, 1, 0).astype(jnp.int32)
            return gx * 16 + cnt

        starts = (
            [jnp.int32(0)]
            + [lower_bound(jnp.int32(s)) for s in range(1, _B)]
            + [jnp.int32(_CHUNK)]
        )

        zv = jnp.zeros((16,), jnp.float32)
        for s in range(_B):
            for j in range(_CG):
                acc[s, pl.ds(j * 16, 16)] = zv

        def process(buf, blk_row):
            for s in range(_B):
                lo = jnp.clip(starts[s] - blk_row, 0, _R)
                hi = jnp.clip(starts[s + 1] - blk_row, 0, _R)

                def rbody(i, a):
                    return tuple(
                        a[j] + buf[i, pl.ds(j * 16, 16)] for j in range(_CG)
                    )

                a0 = tuple(acc[s, pl.ds(j * 16, 16)] for j in range(_CG))
                a = lax.fori_loop(lo, hi, rbody, a0)
                for j in range(_CG):
                    acc[s, pl.ds(j * 16, 16)] = a[j]

        def gbody(g, carry):
            b0 = 2 * g
            r0 = base + b0 * _R
            pltpu.make_async_copy(
                feats_hbm.at[pl.ds(r0, _R), :], buf0, sem0).wait()
            process(buf0, b0 * _R)

            @pl.when(b0 + 2 < _NBLK)
            def _():
                pltpu.async_copy(
                    feats_hbm.at[pl.ds(r0 + 2 * _R, _R), :], buf0, sem0)

            pltpu.make_async_copy(
                feats_hbm.at[pl.ds(r0 + _R, _R), :], buf1, sem1).wait()
            process(buf1, (b0 + 1) * _R)

            @pl.when(b0 + 3 < _NBLK)
            def _():
                pltpu.async_copy(
                    feats_hbm.at[pl.ds(r0 + 3 * _R, _R), :], buf1, sem1)

            return carry

        lax.fori_loop(0, _NBLK // 2, gbody, jnp.int32(0))

        for s in range(_B):
            c = (starts[s + 1] - starts[s]).astype(jnp.float32)
            vec = jnp.full((16,), c, jnp.float32)
            for j in range(_CG):
                cntb[s, pl.ds(j * 16, 16)] = vec

        pltpu.sync_copy(acc, psum_hbm.at[wid])
        pltpu.sync_copy(cntb, pcnt_hbm.at[wid])

    return k(feats, bidx)


def _combine_body(ps_ref, cs_ref, o_ref):
    s = jnp.sum(ps_ref[...], axis=0)
    c = jnp.sum(cs_ref[...], axis=0)
    o_ref[...] = s / jnp.maximum(c, 1.0)


def _combine(psum, pcnt):
    return pl.pallas_call(
        _combine_body,
        out_shape=jax.ShapeDtypeStruct((_B, _C), jnp.float32),
    )(psum, pcnt)


def kernel(feats, bidx):
    psum, pcnt = _sc_partial(feats, bidx)
    return _combine(psum, pcnt)
